# baseline (speedup 1.0000x reference)
import functools

import jax
import jax.numpy as jnp
from jax import lax
from jax.experimental import pallas as pl
from jax.experimental.pallas import tpu as pltpu
from jax.experimental.pallas import tpu_sc as plsc

_NC = 2
_NS = 16
_NW = _NC * _NS
_CHUNK = 128
_L = 16


def _sc_mesh_kernel(out_shape, scratch):
  mesh = plsc.VectorSubcoreMesh(core_axis_name="c", subcore_axis_name="s")
  return functools.partial(
      pl.kernel, mesh=mesh, out_type=out_shape, scratch_types=scratch)


def _sc_gather_scatter_s(x, src_p, dst_p, z_s):
  n, d = x.shape
  e_pad = src_p.shape[0]
  ept = e_pad // _NW
  nchunk = ept // _CHUNK
  npad = z_s.shape[0]
  zrows = npad // _NS

  @_sc_mesh_kernel(
      jax.ShapeDtypeStruct((_NC, npad, d), jnp.float32),
      [
          pltpu.VMEM((_CHUNK,), jnp.int32),
          pltpu.VMEM((_CHUNK,), jnp.int32),
          pltpu.VMEM((_CHUNK, d), jnp.float32),
          pltpu.VMEM_SHARED((npad, d), jnp.float32),
          pltpu.SemaphoreType.DMA,
      ])
  def body(x_hbm, src_hbm, dst_hbm, zs_hbm, s_out, idx_v, dst_v, rows_v,
           acc_s, sem):
    c = lax.axis_index("c")
    s = lax.axis_index("s")
    wid = c * _NS + s

    pltpu.sync_copy(zs_hbm.at[pl.ds(s * zrows, zrows)],
                    acc_s.at[pl.ds(s * zrows, zrows)])
    plsc.subcore_barrier()

    def chunk(k, carry):
      off = wid * ept + k * _CHUNK
      pltpu.sync_copy(src_hbm.at[pl.ds(off, _CHUNK)], idx_v)
      pltpu.sync_copy(dst_hbm.at[pl.ds(off, _CHUNK)], dst_v)
      pltpu.async_copy(x_hbm.at[idx_v], rows_v, sem).wait()
      pltpu.sync_copy(rows_v, acc_s.at[dst_v], add=True)
      return carry

    lax.fori_loop(0, nchunk, chunk, 0)
    plsc.subcore_barrier()

    pltpu.sync_copy(acc_s.at[pl.ds(s * zrows, zrows)],
                    s_out.at[c, pl.ds(s * zrows, zrows)])

  return body(x, src_p, dst_p, z_s)


def _sc_scatter_t(eapk, dst_p, z_s, de):
  e_pad = dst_p.shape[0]
  ept = e_pad // _NW
  nchunk = ept // _CHUNK
  npad = z_s.shape[0]
  zrows = npad // _NS
  perrow = 128 // de
  krows = _CHUNK // perrow

  @_sc_mesh_kernel(
      jax.ShapeDtypeStruct((_NC, npad, 128), jnp.float32),
      [
          pltpu.VMEM((_CHUNK,), jnp.int32),
          pltpu.VMEM((krows, 128), jnp.float32),
          pltpu.VMEM((_CHUNK, 128), jnp.float32),
          pltpu.VMEM_SHARED((npad, 128), jnp.float32),
          pltpu.SemaphoreType.DMA,
      ])
  def body(eapk_hbm, dst_hbm, zs_hbm, t_out, dst_v, eapk_v, ea128_v,
           acc_t, sem):
    c = lax.axis_index("c")
    s = lax.axis_index("s")
    wid = c * _NS + s

    def zrow(i, carry):
      for j in range(perrow):
        ea128_v[i, pl.ds(j * _L, _L)] = jnp.zeros((_L,), jnp.float32)
      return carry
    lax.fori_loop(0, _CHUNK, zrow, 0)

    pltpu.sync_copy(zs_hbm.at[pl.ds(s * zrows, zrows)],
                    acc_t.at[pl.ds(s * zrows, zrows)])
    plsc.subcore_barrier()

    def chunk(k, carry):
      off = wid * ept + k * _CHUNK
      pltpu.sync_copy(dst_hbm.at[pl.ds(off, _CHUNK)], dst_v)
      pltpu.sync_copy(
          eapk_hbm.at[pl.ds(wid * (ept // perrow) + k * krows, krows)],
          eapk_v)
      for t in range(krows):
        for j in range(perrow):
          ea128_v[t * perrow + j, pl.ds(0, de)] = (
              eapk_v[t, pl.ds(j * de, de)])
      pltpu.sync_copy(ea128_v, acc_t.at[dst_v], add=True)
      return carry

    lax.fori_loop(0, nchunk, chunk, 0)
    plsc.subcore_barrier()

    pltpu.sync_copy(acc_t.at[pl.ds(s * zrows, zrows)],
                    t_out.at[c, pl.ds(s * zrows, zrows)])

  return body(eapk, dst_p, z_s)


def _tc_dense(x, s_parts, t_parts, batch3, w_self, w_msg, w_edge_p, b_msg2,
              wp_t, bp2, *, bn, g):
  n, d = x.shape
  h = w_self.shape[1]
  nb = n // bn

  tw = t_parts.shape[2]

  def body(x_ref, s_ref, t_ref, b_ref, ws_ref, wm_ref, we_ref, bm_ref,
           wp_ref, bp_ref, out_ref, gsum_ref, cnt_ref):
    i = pl.program_id(0)

    @pl.when(i == 0)
    def _():
      gsum_ref[...] = jnp.zeros_like(gsum_ref)
      cnt_ref[...] = jnp.zeros_like(cnt_ref)

    xb = x_ref[...]
    sb = s_ref[0] + s_ref[1]
    tb = t_ref[0] + t_ref[1]
    def split_dot(v, w_ref):
      v_hi = v.astype(jnp.bfloat16).astype(jnp.float32)
      return (jnp.dot(v_hi, w_ref[...], preferred_element_type=jnp.float32)
              + jnp.dot(v - v_hi, w_ref[...],
                        preferred_element_type=jnp.float32))

    hb = (jnp.dot(xb, ws_ref[...], preferred_element_type=jnp.float32)
          + split_dot(sb, wm_ref) + split_dot(tb, we_ref) + bm_ref[...])
    hb = jnp.where(hb > 0, hb, 0.01 * hb)

    bvals = b_ref[...].reshape(1, bn)
    seg = lax.broadcasted_iota(jnp.int32, (g, bn), 0)
    oneh = (seg == bvals).astype(jnp.float32)
    hb_hi = hb.astype(jnp.bfloat16).astype(jnp.float32)
    hb_lo = hb - hb_hi
    gsum_ref[...] += (
        jnp.dot(oneh, hb_hi, preferred_element_type=jnp.float32)
        + jnp.dot(oneh, hb_lo, preferred_element_type=jnp.float32))
    cnt_ref[...] += jnp.sum(oneh, axis=1, keepdims=True)

    @pl.when(i == nb - 1)
    def _():
      gmean = gsum_ref[...] / jnp.maximum(cnt_ref[...], 1.0)
      nrm = jnp.sqrt(jnp.sum(gmean * gmean, axis=1, keepdims=True))
      embs = gmean / jnp.maximum(nrm, 1e-12)
      embs_r = embs.astype(jnp.bfloat16).astype(jnp.float32)
      wp_r = wp_ref[...].astype(jnp.bfloat16).astype(jnp.float32)
      val = jnp.sum(embs_r * wp_r, axis=1, keepdims=True)
      out_ref[...] = val + bp_ref[...]

  return pl.pallas_call(
      body,
      grid=(nb,),
      in_specs=[
          pl.BlockSpec((bn, d), lambda i: (i, 0)),
          pl.BlockSpec((_NC, bn, d), lambda i: (0, i, 0)),
          pl.BlockSpec((_NC, bn, tw), lambda i: (0, i, 0)),
          pl.BlockSpec((1, 1, bn), lambda i: (i, 0, 0)),
          pl.BlockSpec((d, h), lambda i: (0, 0)),
          pl.BlockSpec((d, h), lambda i: (0, 0)),
          pl.BlockSpec((tw, h), lambda i: (0, 0)),
          pl.BlockSpec((1, h), lambda i: (0, 0)),
          pl.BlockSpec((1, h), lambda i: (0, 0)),
          pl.BlockSpec((1, h), lambda i: (0, 0)),
      ],
      out_specs=pl.BlockSpec((g, h), lambda i: (0, 0)),
      out_shape=jax.ShapeDtypeStruct((g, h), jnp.float32),
      scratch_shapes=[
          pltpu.VMEM((g, h), jnp.float32),
          pltpu.VMEM((g, 1), jnp.float32),
      ],
      compiler_params=pltpu.CompilerParams(
          dimension_semantics=("arbitrary",)),
  )(x, s_parts, t_parts, batch3, w_self, w_msg, w_edge_p, b_msg2, wp_t, bp2)


def kernel(x, edge_index, edge_attr, batch, W_self, W_msg, W_edge, b_msg,
           Wp, bp):
  n, d = x.shape
  e = edge_index.shape[1]
  de = edge_attr.shape[1]
  h = W_self.shape[1]
  g = 64

  per_tile = -(-e // (_NW * 2 * _CHUNK)) * (2 * _CHUNK)
  e_pad = per_tile * _NW
  pad = e_pad - e
  src_p = jnp.concatenate([edge_index[0], jnp.zeros((pad,), jnp.int32)])
  dst_p = jnp.concatenate([edge_index[1], jnp.full((pad,), n, jnp.int32)])
  x_r = x.astype(jnp.bfloat16).astype(jnp.float32)
  ea_r = edge_attr.astype(jnp.bfloat16).astype(jnp.float32)
  ea_p = jnp.concatenate(
      [ea_r, jnp.zeros((pad, de), jnp.float32)], axis=0)
  eapk = ea_p.reshape(e_pad * de // 128, 128)

  npad = -(-(n + 1) // (_NS * 8)) * (_NS * 8)
  z_s = jnp.zeros((npad, d), jnp.float32)

  s_parts = _sc_gather_scatter_s(x_r, src_p, dst_p, z_s)
  t_parts = _sc_scatter_t(eapk, dst_p, z_s, de)
  tw = t_parts.shape[2]

  bn = 400
  batch3 = batch.reshape(n // bn, 1, bn)
  b_msg2 = b_msg.reshape(1, h)
  wp_t = Wp.reshape(1, h)
  bp2 = jnp.broadcast_to(bp.reshape(1, 1), (1, h))
  w_edge_p = jnp.zeros((tw, h), jnp.float32).at[:de].set(W_edge)

  out_full = _tc_dense(x, s_parts, t_parts, batch3, W_self, W_msg,
                       w_edge_p, b_msg2, wp_t, bp2, bn=bn, g=g)
  return out_full[:, :1]


# speedup vs baseline: 1.4439x; 1.4439x over previous
import functools

import jax
import jax.numpy as jnp
from jax import lax
from jax.experimental import pallas as pl
from jax.experimental.pallas import tpu as pltpu
from jax.experimental.pallas import tpu_sc as plsc

_NC = 2
_NS = 16
_NW = _NC * _NS
_CHUNK = 128
_L = 16


def _sc_mesh_kernel(out_shape, scratch):
  mesh = plsc.VectorSubcoreMesh(core_axis_name="c", subcore_axis_name="s")
  return functools.partial(
      pl.kernel, mesh=mesh, out_type=out_shape, scratch_types=scratch)


def _sc_gather_scatter_s(x, src_p, dst_p, z_s):
  n, d = x.shape
  e_pad = src_p.shape[0]
  ept = e_pad // _NW
  nchunk = ept // _CHUNK
  npad = z_s.shape[0]
  zrows = npad // _NS

  @_sc_mesh_kernel(
      jax.ShapeDtypeStruct((_NC, npad, d), jnp.float32),
      [
          pltpu.VMEM((_CHUNK,), jnp.int32),
          pltpu.VMEM((_CHUNK,), jnp.int32),
          pltpu.VMEM((_CHUNK, d), jnp.float32),
          pltpu.VMEM_SHARED((npad, d), jnp.float32),
          pltpu.SemaphoreType.DMA,
      ])
  def body(x_hbm, src_hbm, dst_hbm, zs_hbm, s_out, idx_v, dst_v, rows_v,
           acc_s, sem):
    c = lax.axis_index("c")
    s = lax.axis_index("s")
    wid = c * _NS + s

    pltpu.sync_copy(zs_hbm.at[pl.ds(s * zrows, zrows)],
                    acc_s.at[pl.ds(s * zrows, zrows)])
    plsc.subcore_barrier()

    def chunk(k, carry):
      off = wid * ept + k * _CHUNK
      pltpu.sync_copy(src_hbm.at[pl.ds(off, _CHUNK)], idx_v)
      pltpu.sync_copy(dst_hbm.at[pl.ds(off, _CHUNK)], dst_v)
      pltpu.async_copy(x_hbm.at[idx_v], rows_v, sem).wait()
      pltpu.sync_copy(rows_v, acc_s.at[dst_v], add=True)
      return carry

    lax.fori_loop(0, nchunk, chunk, 0)
    plsc.subcore_barrier()

    pltpu.sync_copy(acc_s.at[pl.ds(s * zrows, zrows)],
                    s_out.at[c, pl.ds(s * zrows, zrows)])

  return body(x, src_p, dst_p, z_s)


def _sc_scatter_t(eapk, dst_p, z_s, de):
  e_pad = dst_p.shape[0]
  ept = e_pad // _NW
  nchunk = ept // _CHUNK
  npad = z_s.shape[0]
  zrows = npad // _NS
  perrow = 128 // de
  krows = _CHUNK // perrow

  @_sc_mesh_kernel(
      jax.ShapeDtypeStruct((_NC, npad, 128), jnp.float32),
      [
          pltpu.VMEM((_CHUNK,), jnp.int32),
          pltpu.VMEM((krows, 128), jnp.float32),
          pltpu.VMEM((_CHUNK, 128), jnp.float32),
          pltpu.VMEM_SHARED((npad, 128), jnp.float32),
          pltpu.SemaphoreType.DMA,
      ])
  def body(eapk_hbm, dst_hbm, zs_hbm, t_out, dst_v, eapk_v, ea128_v,
           acc_t, sem):
    c = lax.axis_index("c")
    s = lax.axis_index("s")
    wid = c * _NS + s

    def zrow(i, carry):
      for j in range(perrow):
        ea128_v[i, pl.ds(j * _L, _L)] = jnp.zeros((_L,), jnp.float32)
      return carry
    lax.fori_loop(0, _CHUNK, zrow, 0)

    pltpu.sync_copy(zs_hbm.at[pl.ds(s * zrows, zrows)],
                    acc_t.at[pl.ds(s * zrows, zrows)])
    plsc.subcore_barrier()

    def chunk(k, carry):
      off = wid * ept + k * _CHUNK
      pltpu.sync_copy(dst_hbm.at[pl.ds(off, _CHUNK)], dst_v)
      pltpu.sync_copy(
          eapk_hbm.at[pl.ds(wid * (ept // perrow) + k * krows, krows)],
          eapk_v)
      for t in range(krows):
        for j in range(perrow):
          ea128_v[t * perrow + j, pl.ds(0, de)] = (
              eapk_v[t, pl.ds(j * de, de)])
      pltpu.sync_copy(ea128_v, acc_t.at[dst_v], add=True)
      return carry

    lax.fori_loop(0, nchunk, chunk, 0)
    plsc.subcore_barrier()

    pltpu.sync_copy(acc_t.at[pl.ds(s * zrows, zrows)],
                    t_out.at[c, pl.ds(s * zrows, zrows)])

  return body(eapk, dst_p, z_s)


def _tc_dense(x, s_parts, t_parts, batch3, w_self, w_msg, w_edge_p, b_msg2,
              wp_t, bp2, *, bn, g):
  n, d = x.shape
  h = w_self.shape[1]
  nb = n // bn

  tw = t_parts.shape[2]

  def body(x_ref, s_ref, t_ref, b_ref, ws_ref, wm_ref, we_ref, bm_ref,
           wp_ref, bp_ref, out_ref, gsum_ref, cnt_ref):
    i = pl.program_id(0)

    @pl.when(i == 0)
    def _():
      gsum_ref[...] = jnp.zeros_like(gsum_ref)
      cnt_ref[...] = jnp.zeros_like(cnt_ref)

    xb = x_ref[...]
    sb = s_ref[0] + s_ref[1]
    tb = t_ref[0] + t_ref[1]
    def split_dot(v, w_ref):
      v_hi = v.astype(jnp.bfloat16).astype(jnp.float32)
      return (jnp.dot(v_hi, w_ref[...], preferred_element_type=jnp.float32)
              + jnp.dot(v - v_hi, w_ref[...],
                        preferred_element_type=jnp.float32))

    hb = (jnp.dot(xb, ws_ref[...], preferred_element_type=jnp.float32)
          + split_dot(sb, wm_ref) + split_dot(tb, we_ref) + bm_ref[...])
    hb = jnp.where(hb > 0, hb, 0.01 * hb)

    bvals = b_ref[...].reshape(1, bn)
    seg = lax.broadcasted_iota(jnp.int32, (g, bn), 0)
    oneh = (seg == bvals).astype(jnp.float32)
    hb_hi = hb.astype(jnp.bfloat16).astype(jnp.float32)
    hb_lo = hb - hb_hi
    gsum_ref[...] += (
        jnp.dot(oneh, hb_hi, preferred_element_type=jnp.float32)
        + jnp.dot(oneh, hb_lo, preferred_element_type=jnp.float32))
    cnt_ref[...] += jnp.sum(oneh, axis=1, keepdims=True)

    @pl.when(i == nb - 1)
    def _():
      gmean = gsum_ref[...] / jnp.maximum(cnt_ref[...], 1.0)
      nrm = jnp.sqrt(jnp.sum(gmean * gmean, axis=1, keepdims=True))
      embs = gmean / jnp.maximum(nrm, 1e-12)
      embs_r = embs.astype(jnp.bfloat16).astype(jnp.float32)
      wp_r = wp_ref[...].astype(jnp.bfloat16).astype(jnp.float32)
      val = jnp.sum(embs_r * wp_r, axis=1, keepdims=True)
      out_ref[...] = val + bp_ref[...]

  return pl.pallas_call(
      body,
      grid=(nb,),
      in_specs=[
          pl.BlockSpec((bn, d), lambda i: (i, 0)),
          pl.BlockSpec((_NC, bn, d), lambda i: (0, i, 0)),
          pl.BlockSpec((_NC, bn, tw), lambda i: (0, i, 0)),
          pl.BlockSpec((1, 1, bn), lambda i: (i, 0, 0)),
          pl.BlockSpec((d, h), lambda i: (0, 0)),
          pl.BlockSpec((d, h), lambda i: (0, 0)),
          pl.BlockSpec((tw, h), lambda i: (0, 0)),
          pl.BlockSpec((1, h), lambda i: (0, 0)),
          pl.BlockSpec((1, h), lambda i: (0, 0)),
          pl.BlockSpec((1, h), lambda i: (0, 0)),
      ],
      out_specs=pl.BlockSpec((g, h), lambda i: (0, 0)),
      out_shape=jax.ShapeDtypeStruct((g, h), jnp.float32),
      scratch_shapes=[
          pltpu.VMEM((g, h), jnp.float32),
          pltpu.VMEM((g, 1), jnp.float32),
      ],
      compiler_params=pltpu.CompilerParams(
          dimension_semantics=("arbitrary",)),
  )(x, s_parts, t_parts, batch3, w_self, w_msg, w_edge_p, b_msg2, wp_t, bp2)


def kernel(x, edge_index, edge_attr, batch, W_self, W_msg, W_edge, b_msg,
           Wp, bp):
  n, d = x.shape
  e = edge_index.shape[1]
  de = edge_attr.shape[1]
  h = W_self.shape[1]
  g = 64

  per_tile = -(-e // (_NW * _CHUNK)) * _CHUNK
  e_pad = per_tile * _NW
  pad = e_pad - e
  npad = -(-(n + 1) // (_NS * 8)) * (_NS * 8)
  src_p = jnp.concatenate([edge_index[0], jnp.zeros((pad,), jnp.int32)])
  dst_pad = n + jnp.arange(pad, dtype=jnp.int32) % (npad - n)
  dst_p = jnp.concatenate([edge_index[1], dst_pad])
  x_r = x.astype(jnp.bfloat16).astype(jnp.float32)
  ea_r = edge_attr.astype(jnp.bfloat16).astype(jnp.float32)
  ea_p = jnp.concatenate(
      [ea_r, jnp.zeros((pad, de), jnp.float32)], axis=0)
  eapk = ea_p.reshape(e_pad * de // 128, 128)
  z_s = jnp.zeros((npad, d), jnp.float32)

  s_parts = _sc_gather_scatter_s(x_r, src_p, dst_p, z_s)
  t_parts = _sc_scatter_t(eapk, dst_p, z_s, de)
  tw = t_parts.shape[2]

  bn = 400
  batch3 = batch.reshape(n // bn, 1, bn)
  b_msg2 = b_msg.reshape(1, h)
  wp_t = Wp.reshape(1, h)
  bp2 = jnp.broadcast_to(bp.reshape(1, 1), (1, h))
  w_edge_p = jnp.zeros((tw, h), jnp.float32).at[:de].set(W_edge)

  out_full = _tc_dense(x, s_parts, t_parts, batch3, W_self, W_msg,
                       w_edge_p, b_msg2, wp_t, bp2, bn=bn, g=g)
  return out_full[:, :1]
